# baseline (speedup 1.0000x reference)
import functools

import jax
import jax.numpy as jnp
from jax import lax
from jax.experimental import pallas as pl
from jax.experimental.pallas import tpu as pltpu
from jax.experimental.pallas import tpu_sc as plsc

B = 128
T = 100001
O = 50000
F = 16

NC = 2
NS = 16
NW = NC * NS
L = 16

CH = 8
CPW = 200
NCHUNKS = NW * CPW
OPAD = NCHUNKS * CH
NVR = B // L
BPAD = 16

_GATHER_DNUMS = lax.GatherDimensionNumbers(
    offset_dims=(), collapsed_slice_dims=(0,), start_index_map=(0,))


def _lane_bcast(vec, lane):
    idx = jnp.full((L, 1), lane, dtype=jnp.int32)
    return lax.gather(vec, idx, _GATHER_DNUMS, (1,),
                      mode=lax.GatherScatterMode.PROMISE_IN_BOUNDS)


def _compute_chunk(w_stage, bias_stage, rows_v, out_v):
    brow = bias_stage[pl.ds(0, L)]
    for j in range(CH):
        bj = _lane_bcast(brow, j)
        wrow = w_stage[pl.ds(j * F, F)]
        accs = [bj] * NVR
        for f in range(F):
            e = j * F + f
            wv = _lane_bcast(wrow, f)
            for v in range(NVR):
                r = rows_v[e, pl.ds(v * L, L)]
                accs[v] = accs[v] + wv * r
        for v in range(NVR):
            out_v[j, pl.ds(v * L, L)] = jnp.maximum(accs[v], 0.0)


def _sc_body(tapeT_hbm, idx_hbm, w_hbm, bias_hbm, out_hbm,
             idx_c, w_c, bias_c, rows_v, out_v, gsem):
    wid = lax.axis_index("s") * NC + lax.axis_index("c")

    def body(k, carry):
        c = wid + k * NW
        pltpu.sync_copy(idx_hbm.at[c], idx_c)
        pltpu.sync_copy(w_hbm.at[c], w_c)
        pltpu.sync_copy(bias_hbm.at[pl.ds(c * CH, L)], bias_c)
        pltpu.async_copy(tapeT_hbm.at[idx_c], rows_v, gsem).wait()
        _compute_chunk(w_c, bias_c, rows_v, out_v)
        pltpu.sync_copy(out_v,
                        out_hbm.at[pl.ds(pl.multiple_of(c * CH, 8), CH)])
        return carry

    lax.fori_loop(0, CPW, body, None)


@functools.partial(
    pl.kernel,
    mesh=plsc.VectorSubcoreMesh(core_axis_name="c", subcore_axis_name="s"),
    out_type=jax.ShapeDtypeStruct((OPAD, B), jnp.float32),
    scratch_types=[
        pltpu.VMEM((CH * F,), jnp.int32),
        pltpu.VMEM((CH * F,), jnp.float32),
        pltpu.VMEM((L,), jnp.float32),
        pltpu.VMEM((CH * F, B), jnp.float32),
        pltpu.VMEM((CH, B), jnp.float32),
        pltpu.SemaphoreType.DMA,
    ],
)
def _sc_kernel(tapeT_hbm, idx_hbm, w_hbm, bias_hbm, out_hbm,
               idx_c, w_c, bias_c, rows_v, out_v, gsem):
    _sc_body(tapeT_hbm, idx_hbm, w_hbm, bias_hbm, out_hbm,
             idx_c, w_c, bias_c, rows_v, out_v, gsem)


@jax.jit
def kernel(tape, weights, bias, input_indices, output_indices):
    tapeT = tape[:, :O].T
    pad = OPAD - O
    idx = jnp.pad(input_indices.astype(jnp.int32),
                  ((0, pad), (0, 0))).reshape(NCHUNKS, CH * F)
    wts = jnp.pad(weights, ((0, pad), (0, 0))).reshape(NCHUNKS, CH * F)
    b = jnp.pad(bias, (0, pad + BPAD))
    outT = _sc_kernel(tapeT, idx, wts, b)
    return jnp.concatenate([tape[:, :O + 1], outT[:O].T], axis=1)


# speedup vs baseline: 1.1743x; 1.1743x over previous
import functools

import jax
import jax.numpy as jnp
from jax import lax
from jax.experimental import pallas as pl
from jax.experimental.pallas import tpu as pltpu
from jax.experimental.pallas import tpu_sc as plsc

B = 128
T = 100001
O = 50000
F = 16

NC = 2
NS = 16
NW = NC * NS
L = 16

CH = 8
CPW = 200
NCHUNKS = NW * CPW
OPAD = NCHUNKS * CH
NVR = B // L
BPAD = 64

_GATHER_DNUMS = lax.GatherDimensionNumbers(
    offset_dims=(), collapsed_slice_dims=(0,), start_index_map=(0,))


def _lane_bcast(vec, lane):
    idx = jnp.full((L, 1), lane, dtype=jnp.int32)
    return lax.gather(vec, idx, _GATHER_DNUMS, (1,),
                      mode=lax.GatherScatterMode.PROMISE_IN_BOUNDS)


def _compute_chunk(k, w_stage, bias_stage, rows_v, out_v):
    brow = bias_stage[pl.ds(k * CH, L)]

    @plsc.parallel_loop(0, CH, unroll=CH)
    def _node(j):
        bj = _lane_bcast(brow, j)
        wrow = w_stage[k, pl.ds(j * F, F)]
        accs = [bj] * NVR
        for f in range(F):
            wv = _lane_bcast(wrow, f)
            for v in range(NVR):
                r = rows_v[j * F + f, pl.ds(v * L, L)]
                accs[v] = accs[v] + wv * r
        for v in range(NVR):
            out_v[j, pl.ds(v * L, L)] = jnp.maximum(accs[v], 0.0)


def _sc_body(tapeT_hbm, idx_hbm, w_hbm, bias_hbm, out_hbm,
             idx_stage, w_stage, bias_stage,
             rows_v0, rows_v1, out_v0, out_v1,
             gsem0, gsem1, osem0, osem1):
    wid = lax.axis_index("s") * NC + lax.axis_index("c")
    base = pl.multiple_of(wid * CPW, 8)

    pltpu.sync_copy(idx_hbm.at[pl.ds(base, CPW)], idx_stage)
    pltpu.sync_copy(w_hbm.at[pl.ds(base, CPW)], w_stage)
    pltpu.sync_copy(bias_hbm.at[pl.ds(base * CH, CPW * CH)],
                    bias_stage.at[pl.ds(0, CPW * CH)])

    slots = ((rows_v0, gsem0, out_v0, osem0),
             (rows_v1, gsem1, out_v1, osem1))

    pltpu.async_copy(tapeT_hbm.at[idx_stage.at[0]], rows_v0, gsem0)
    pltpu.async_copy(tapeT_hbm.at[idx_stage.at[1]], rows_v1, gsem1)

    def body(t, carry):
        for par, (rows_v, gsem, out_v, osem) in enumerate(slots):
            k = 2 * t + par
            c = base + k
            pltpu.make_async_copy(
                tapeT_hbm.at[pl.ds(0, CH * F)], rows_v, gsem).wait()

            @pl.when(t > 0)
            def _():
                pltpu.make_async_copy(
                    out_hbm.at[pl.ds(0, CH)], out_v, osem).wait()

            _compute_chunk(k, w_stage, bias_stage, rows_v, out_v)

            @pl.when(k + 2 < CPW)
            def _():
                pltpu.async_copy(
                    tapeT_hbm.at[idx_stage.at[k + 2]], rows_v, gsem)

            pltpu.async_copy(
                out_v, out_hbm.at[pl.ds(pl.multiple_of(c * CH, 8), CH)], osem)
        return carry

    lax.fori_loop(0, CPW // 2, body, None)

    pltpu.make_async_copy(out_hbm.at[pl.ds(0, CH)], out_v0, osem0).wait()
    pltpu.make_async_copy(out_hbm.at[pl.ds(0, CH)], out_v1, osem1).wait()


@functools.partial(
    pl.kernel,
    mesh=plsc.VectorSubcoreMesh(core_axis_name="c", subcore_axis_name="s"),
    out_type=jax.ShapeDtypeStruct((OPAD, B), jnp.float32),
    scratch_types=[
        pltpu.VMEM((CPW, CH * F), jnp.int32),
        pltpu.VMEM((CPW, CH * F), jnp.float32),
        pltpu.VMEM((CPW * CH + BPAD,), jnp.float32),
        pltpu.VMEM((CH * F, B), jnp.float32),
        pltpu.VMEM((CH * F, B), jnp.float32),
        pltpu.VMEM((CH, B), jnp.float32),
        pltpu.VMEM((CH, B), jnp.float32),
        pltpu.SemaphoreType.DMA,
        pltpu.SemaphoreType.DMA,
        pltpu.SemaphoreType.DMA,
        pltpu.SemaphoreType.DMA,
    ],
)
def _sc_kernel(tapeT_hbm, idx_hbm, w_hbm, bias_hbm, out_hbm,
               idx_stage, w_stage, bias_stage,
               rows_v0, rows_v1, out_v0, out_v1,
               gsem0, gsem1, osem0, osem1):
    _sc_body(tapeT_hbm, idx_hbm, w_hbm, bias_hbm, out_hbm,
             idx_stage, w_stage, bias_stage,
             rows_v0, rows_v1, out_v0, out_v1,
             gsem0, gsem1, osem0, osem1)


@jax.jit
def kernel(tape, weights, bias, input_indices, output_indices):
    tapeT = tape[:, :O].T
    pad = OPAD - O
    idx = jnp.pad(input_indices.astype(jnp.int32),
                  ((0, pad), (0, 0))).reshape(NCHUNKS, CH * F)
    wts = jnp.pad(weights, ((0, pad), (0, 0))).reshape(NCHUNKS, CH * F)
    b = jnp.pad(bias, (0, pad))
    outT = _sc_kernel(tapeT, idx, wts, b)
    return jnp.concatenate([tape[:, :O + 1], outT[:O].T], axis=1)
